# Initial kernel scaffold; baseline (speedup 1.0000x reference)
import jax
import jax.numpy as jnp
from jax.experimental import pallas as pl


def kernel(f_atoms, f_bonds, W_i, W_h, W_o, b_o, a2b, b2a, b2revb, n_mols):
    raise NotImplementedError("write your pallas kernel here")


# speedup vs baseline: 1.0712x; 1.0712x over previous
import functools

import jax
import jax.numpy as jnp
from jax import lax
from jax.experimental import pallas as pl
from jax.experimental.pallas import tpu as pltpu
from jax.experimental.pallas import tpu_sc as plsc

F32 = jnp.float32

NB = 320000
NA = 10000
MAXNB = 32
AF = 128
H = 50
HP = 64
DEPTH = 3
MOLS = 500
A_SIZE = NA // MOLS
B_SIZE = NB // MOLS

NW = 32
APW = 320
NAP = NW * APW
A_CHUNK = 8
BPW = NB // NW
B_CHUNK = 80

_mesh = plsc.VectorSubcoreMesh(core_axis_name="c", subcore_axis_name="s")
_SC_PARAMS = pltpu.CompilerParams(use_tc_tiling_on_sc=False)


def _wid():
    return lax.axis_index("s") * 2 + lax.axis_index("c")


@functools.partial(
    pl.kernel,
    out_type=jax.ShapeDtypeStruct((NAP, HP), F32),
    mesh=_mesh,
    compiler_params=_SC_PARAMS,
    scratch_types=[
        pltpu.VMEM((128,), jnp.int32),
        pltpu.VMEM((128,), jnp.int32),
        pltpu.VMEM((128, HP), F32),
        pltpu.VMEM((128, HP), F32),
        pltpu.VMEM((A_CHUNK, HP), F32),
        pltpu.SemaphoreType.DMA,
        pltpu.SemaphoreType.DMA,
    ],
)
def _seg_sum(table_hbm, a2b_hbm, h_hbm, idx0_v, idx1_v, rows0_v, rows1_v,
             acc_v, sem0, sem1):
    wid = _wid()
    abase = wid * APW
    ibase = abase * MAXNB

    def chunk(c, carry):
        ioff = ibase + c * (A_CHUNK * MAXNB)
        pltpu.sync_copy(a2b_hbm.at[pl.ds(ioff, 128)], idx0_v)
        pltpu.sync_copy(a2b_hbm.at[pl.ds(ioff + 128, 128)], idx1_v)
        cp0 = pltpu.async_copy(table_hbm.at[idx0_v], rows0_v, sem0)
        cp1 = pltpu.async_copy(table_hbm.at[idx1_v], rows1_v, sem1)
        cp0.wait()
        cp1.wait()
        for a in range(A_CHUNK):
            rows = rows0_v if a < 4 else rows1_v
            r0 = (a % 4) * MAXNB
            for j in range(HP // 16):
                sl = pl.ds(j * 16, 16)
                s = rows[r0, sl]
                for r in range(1, MAXNB):
                    s = s + rows[r0 + r, sl]
                acc_v[a, sl] = s
        pltpu.sync_copy(acc_v, h_hbm.at[pl.ds(abase + c * A_CHUNK, A_CHUNK)])
        return carry

    lax.fori_loop(0, APW // A_CHUNK, chunk, 0)


@functools.partial(
    pl.kernel,
    out_type=jax.ShapeDtypeStruct((NB, HP), F32),
    mesh=_mesh,
    compiler_params=_SC_PARAMS,
    scratch_types=[
        pltpu.VMEM((B_CHUNK,), jnp.int32),
        pltpu.VMEM((B_CHUNK,), jnp.int32),
        pltpu.VMEM((B_CHUNK, HP), F32),
        pltpu.VMEM((B_CHUNK, HP), F32),
        pltpu.VMEM((B_CHUNK, HP), F32),
        pltpu.VMEM((B_CHUNK, HP), F32),
        pltpu.SemaphoreType.DMA,
        pltpu.SemaphoreType.DMA,
    ],
)
def _bond_update(inp_hbm, g_hbm, h_hbm, b2a_hbm, b2revb_hbm, msg_hbm,
                 ia_v, ir_v, ha_v, gr_v, inp_v, out_v, sem0, sem1):
    wid = _wid()
    bbase = wid * BPW

    def chunk(c, carry):
        boff = bbase + c * B_CHUNK
        pltpu.sync_copy(b2a_hbm.at[pl.ds(boff, B_CHUNK)], ia_v)
        pltpu.sync_copy(b2revb_hbm.at[pl.ds(boff, B_CHUNK)], ir_v)
        cpa = pltpu.async_copy(h_hbm.at[ia_v], ha_v, sem0)
        cpr = pltpu.async_copy(g_hbm.at[ir_v], gr_v, sem1)
        pltpu.sync_copy(inp_hbm.at[pl.ds(boff, B_CHUNK)], inp_v)
        cpa.wait()
        cpr.wait()

        def row(r, rcarry):
            for j in range(HP // 16):
                sl = pl.ds(j * 16, 16)
                out_v[r, sl] = jnp.maximum(
                    inp_v[r, sl] + ha_v[r, sl] - gr_v[r, sl], 0.0)
            return rcarry

        lax.fori_loop(0, B_CHUNK, row, 0)
        pltpu.sync_copy(out_v, msg_hbm.at[pl.ds(boff, B_CHUNK)])
        return carry

    lax.fori_loop(0, BPW // B_CHUNK, chunk, 0)


_MM_BLK = 1280


def _mm1_body(fb, wi, wh, inp_ref, g_ref):
    t = jnp.dot(fb[...], wi[...], preferred_element_type=F32)
    inp_ref[...] = t
    g_ref[...] = jnp.dot(jnp.maximum(t, 0.0), wh[...],
                         preferred_element_type=F32)


def _mm2_body(m, wh, g_ref):
    g_ref[...] = jnp.dot(m[...], wh[...], preferred_element_type=F32)


_AB = 200
_MB = 10


def _readout_body(fa, am, woa, wob, bo, smat, out1_ref, out2_ref):
    ah = jnp.maximum(
        jnp.dot(fa[...], woa[...], preferred_element_type=F32)
        + jnp.dot(am[...], wob[...], preferred_element_type=F32)
        + bo[...], 0.0)
    mm = jnp.dot(smat[...], ah, preferred_element_type=F32)
    mm50 = mm[:, :H]
    out1_ref[...] = jnp.broadcast_to(mm50[:, None, :], (_MB, A_SIZE, H))
    out2_ref[...] = jnp.broadcast_to(mm50[:, None, :], (_MB, B_SIZE + 1, H))


def kernel(f_atoms, f_bonds, W_i, W_h, W_o, b_o, a2b, b2a, b2revb, n_mols):
    wi_p = jnp.zeros((AF, HP), F32).at[:, :H].set(W_i)
    wh_p = jnp.zeros((HP, HP), F32).at[:H, :H].set(W_h)
    woa_p = jnp.zeros((AF, HP), F32).at[:, :H].set(W_o[:AF])
    wob_p = jnp.zeros((HP, HP), F32).at[:H, :H].set(W_o[AF:])
    bo_p = jnp.zeros((1, HP), F32).at[0, :H].set(b_o)
    a2b_flat = (jnp.zeros((NAP * MAXNB,), jnp.int32)
                .at[:NA * MAXNB].set(a2b.astype(jnp.int32).reshape(-1)))
    b2a_i = b2a.astype(jnp.int32)
    b2revb_i = b2revb.astype(jnp.int32)
    scale = 1.0 / (f_atoms.shape[0] // n_mols)
    smat = jnp.repeat(jnp.eye(_MB, dtype=F32), A_SIZE, axis=1) * scale

    inp, g = pl.pallas_call(
        _mm1_body,
        grid=(NB // _MM_BLK,),
        in_specs=[
            pl.BlockSpec((_MM_BLK, AF), lambda i: (i, 0)),
            pl.BlockSpec((AF, HP), lambda i: (0, 0)),
            pl.BlockSpec((HP, HP), lambda i: (0, 0)),
        ],
        out_specs=[
            pl.BlockSpec((_MM_BLK, HP), lambda i: (i, 0)),
            pl.BlockSpec((_MM_BLK, HP), lambda i: (i, 0)),
        ],
        out_shape=[
            jax.ShapeDtypeStruct((NB, HP), F32),
            jax.ShapeDtypeStruct((NB, HP), F32),
        ],
    )(f_bonds, wi_p, wh_p)

    msg = None
    for t in range(DEPTH - 1):
        if t > 0:
            g = pl.pallas_call(
                _mm2_body,
                grid=(NB // _MM_BLK,),
                in_specs=[
                    pl.BlockSpec((_MM_BLK, HP), lambda i: (i, 0)),
                    pl.BlockSpec((HP, HP), lambda i: (0, 0)),
                ],
                out_specs=pl.BlockSpec((_MM_BLK, HP), lambda i: (i, 0)),
                out_shape=jax.ShapeDtypeStruct((NB, HP), F32),
            )(msg, wh_p)
        h = _seg_sum(g, a2b_flat)
        msg = _bond_update(inp, g, h, b2a_i, b2revb_i)

    am = _seg_sum(msg, a2b_flat)

    out1, out2 = pl.pallas_call(
        _readout_body,
        grid=(MOLS // _MB,),
        in_specs=[
            pl.BlockSpec((_AB, AF), lambda i: (i, 0)),
            pl.BlockSpec((_AB, HP), lambda i: (i, 0)),
            pl.BlockSpec((AF, HP), lambda i: (0, 0)),
            pl.BlockSpec((HP, HP), lambda i: (0, 0)),
            pl.BlockSpec((1, HP), lambda i: (0, 0)),
            pl.BlockSpec((_MB, _AB), lambda i: (0, 0)),
        ],
        out_specs=[
            pl.BlockSpec((_MB, A_SIZE, H), lambda i: (i, 0, 0)),
            pl.BlockSpec((_MB, B_SIZE + 1, H), lambda i: (i, 0, 0)),
        ],
        out_shape=[
            jax.ShapeDtypeStruct((MOLS, A_SIZE, H), F32),
            jax.ShapeDtypeStruct((MOLS, B_SIZE + 1, H), F32),
        ],
    )(f_atoms, am, woa_p, wob_p, bo_p, smat)

    return (out1.reshape(NA, H), out2.reshape(NB + MOLS, H))


# speedup vs baseline: 1.1824x; 1.1038x over previous
import functools

import jax
import jax.numpy as jnp
from jax import lax
from jax.experimental import pallas as pl
from jax.experimental.pallas import tpu as pltpu
from jax.experimental.pallas import tpu_sc as plsc

F32 = jnp.float32

NB = 320000
NA = 10000
MAXNB = 32
AF = 128
H = 50
HP = 64
DEPTH = 3
MOLS = 500
A_SIZE = NA // MOLS
B_SIZE = NB // MOLS

NW = 32
APW = 320
NAP = NW * APW
BPW = 10240
NBP = NW * BPW
CH = 128
A_CHUNK = CH // MAXNB
NCH_A = APW // A_CHUNK
NCH_B = BPW // CH
NBUF = 2

_mesh = plsc.VectorSubcoreMesh(core_axis_name="c", subcore_axis_name="s")
_SC_PARAMS = pltpu.CompilerParams(use_tc_tiling_on_sc=False)


def _wid():
    return lax.axis_index("s") * 2 + lax.axis_index("c")


@functools.partial(
    pl.kernel,
    out_type=jax.ShapeDtypeStruct((NAP, HP), F32),
    mesh=_mesh,
    compiler_params=_SC_PARAMS,
    scratch_types=[
        pltpu.VMEM((APW * MAXNB,), jnp.int32),
        pltpu.VMEM((NBUF, CH, HP), F32),
        pltpu.VMEM((APW, HP), F32),
        pltpu.SemaphoreType.DMA,
        pltpu.SemaphoreType.DMA,
    ],
)
def _seg_sum(table_hbm, a2b_hbm, h_hbm, idx_v, rows_v, hstage_v, sem0, sem1):
    wid = _wid()
    abase = wid * APW
    sems = (sem0, sem1)
    pltpu.sync_copy(a2b_hbm.at[pl.ds(abase * MAXNB, APW * MAXNB)], idx_v)

    def gather(c, b):
        return pltpu.make_async_copy(
            table_hbm.at[idx_v.at[pl.ds(c * CH, CH)]], rows_v.at[b], sems[b])

    gather(0, 0).start()
    gather(1, 1).start()

    def outer(ci, carry):
        for b in range(NBUF):
            c = ci * NBUF + b
            gather(c, b).wait()
            for a in range(A_CHUNK):
                r0 = a * MAXNB
                for j in range(HP // 16):
                    sl = pl.ds(j * 16, 16)
                    s = rows_v[b, r0, sl]
                    for r in range(1, MAXNB):
                        s = s + rows_v[b, r0 + r, sl]
                    hstage_v[c * A_CHUNK + a, sl] = s

            @pl.when(c + NBUF < NCH_A)
            def _():
                gather(c + NBUF, b).start()
        return carry

    lax.fori_loop(0, NCH_A // NBUF, outer, 0)
    pltpu.sync_copy(hstage_v, h_hbm.at[pl.ds(abase, APW)])


@functools.partial(
    pl.kernel,
    out_type=jax.ShapeDtypeStruct((NBP, HP), F32),
    mesh=_mesh,
    compiler_params=_SC_PARAMS,
    scratch_types=[
        pltpu.VMEM((BPW,), jnp.int32),
        pltpu.VMEM((BPW,), jnp.int32),
        pltpu.VMEM((NBUF, CH, HP), F32),
        pltpu.VMEM((NBUF, CH, HP), F32),
        pltpu.VMEM((NBUF, CH, HP), F32),
        pltpu.VMEM((NBUF, CH, HP), F32),
        [pltpu.SemaphoreType.DMA] * 8,
    ],
)
def _bond_update(inp_hbm, g_hbm, h_hbm, b2a_hbm, b2revb_hbm, msg_hbm,
                 ia_v, ir_v, ha_v, gr_v, inp_v, out_v, sems):
    wid = _wid()
    bbase = wid * BPW
    pltpu.sync_copy(b2a_hbm.at[pl.ds(bbase, BPW)], ia_v)
    pltpu.sync_copy(b2revb_hbm.at[pl.ds(bbase, BPW)], ir_v)

    def in_copies(c, b):
        return (
            pltpu.make_async_copy(
                h_hbm.at[ia_v.at[pl.ds(c * CH, CH)]], ha_v.at[b], sems[b]),
            pltpu.make_async_copy(
                g_hbm.at[ir_v.at[pl.ds(c * CH, CH)]], gr_v.at[b], sems[2 + b]),
            pltpu.make_async_copy(
                inp_hbm.at[pl.ds(bbase + c * CH, CH)], inp_v.at[b],
                sems[4 + b]),
        )

    def out_copy(c, b):
        return pltpu.make_async_copy(
            out_v.at[b], msg_hbm.at[pl.ds(bbase + c * CH, CH)], sems[6 + b])

    for b in range(NBUF):
        for cp in in_copies(b, b):
            cp.start()

    def outer(ci, carry):
        for b in range(NBUF):
            c = ci * NBUF + b
            for cp in in_copies(c, b):
                cp.wait()

            @pl.when(c >= NBUF)
            def _():
                out_copy(c - NBUF, b).wait()

            def row(r, rcarry):
                for j in range(HP // 16):
                    sl = pl.ds(j * 16, 16)
                    out_v[b, r, sl] = jnp.maximum(
                        inp_v[b, r, sl] + ha_v[b, r, sl] - gr_v[b, r, sl],
                        0.0)
                return rcarry

            lax.fori_loop(0, CH, row, 0)
            out_copy(c, b).start()

            @pl.when(c + NBUF < NCH_B)
            def _():
                for cp in in_copies(c + NBUF, b):
                    cp.start()
        return carry

    lax.fori_loop(0, NCH_B // NBUF, outer, 0)
    for b in range(NBUF):
        out_copy(NCH_B - NBUF + b, b).wait()


_MM_BLK = 1280
_MM_LAST = NB // _MM_BLK - 1


def _mm1_body(fb, wi, wh, inp_ref, g_ref):
    t = jnp.dot(fb[...], wi[...], preferred_element_type=F32)
    inp_ref[...] = t
    g_ref[...] = jnp.dot(jnp.maximum(t, 0.0), wh[...],
                         preferred_element_type=F32)


def _mm2_body(m, wh, g_ref):
    g_ref[...] = jnp.dot(m[...], wh[...], preferred_element_type=F32)


_AB = 200
_MB = 10


def _readout_body(fa, am, woa, wob, bo, smat, out1_ref, out2_ref):
    ah = jnp.maximum(
        jnp.dot(fa[...], woa[...], preferred_element_type=F32)
        + jnp.dot(am[...], wob[...], preferred_element_type=F32)
        + bo[...], 0.0)
    mm = jnp.dot(smat[...], ah, preferred_element_type=F32)
    mm50 = mm[:, :H]
    out1_ref[...] = jnp.broadcast_to(mm50[:, None, :], (_MB, A_SIZE, H))
    out2_ref[...] = jnp.broadcast_to(mm50[:, None, :], (_MB, B_SIZE + 1, H))


def kernel(f_atoms, f_bonds, W_i, W_h, W_o, b_o, a2b, b2a, b2revb, n_mols):
    wi_p = jnp.zeros((AF, HP), F32).at[:, :H].set(W_i)
    wh_p = jnp.zeros((HP, HP), F32).at[:H, :H].set(W_h)
    woa_p = jnp.zeros((AF, HP), F32).at[:, :H].set(W_o[:AF])
    wob_p = jnp.zeros((HP, HP), F32).at[:H, :H].set(W_o[AF:])
    bo_p = jnp.zeros((1, HP), F32).at[0, :H].set(b_o)
    a2b_flat = (jnp.zeros((NAP * MAXNB,), jnp.int32)
                .at[:NA * MAXNB].set(a2b.astype(jnp.int32).reshape(-1)))
    b2a_p = jnp.zeros((NBP,), jnp.int32).at[:NB].set(b2a.astype(jnp.int32))
    b2revb_p = (jnp.zeros((NBP,), jnp.int32)
                .at[:NB].set(b2revb.astype(jnp.int32)))
    scale = 1.0 / (f_atoms.shape[0] // n_mols)
    smat = jnp.repeat(jnp.eye(_MB, dtype=F32), A_SIZE, axis=1) * scale

    inp, g = pl.pallas_call(
        _mm1_body,
        grid=(NBP // _MM_BLK,),
        in_specs=[
            pl.BlockSpec((_MM_BLK, AF), lambda i: (jnp.minimum(i, _MM_LAST), 0)),
            pl.BlockSpec((AF, HP), lambda i: (0, 0)),
            pl.BlockSpec((HP, HP), lambda i: (0, 0)),
        ],
        out_specs=[
            pl.BlockSpec((_MM_BLK, HP), lambda i: (i, 0)),
            pl.BlockSpec((_MM_BLK, HP), lambda i: (i, 0)),
        ],
        out_shape=[
            jax.ShapeDtypeStruct((NBP, HP), F32),
            jax.ShapeDtypeStruct((NBP, HP), F32),
        ],
    )(f_bonds, wi_p, wh_p)

    msg = None
    for t in range(DEPTH - 1):
        if t > 0:
            g = pl.pallas_call(
                _mm2_body,
                grid=(NBP // _MM_BLK,),
                in_specs=[
                    pl.BlockSpec((_MM_BLK, HP), lambda i: (i, 0)),
                    pl.BlockSpec((HP, HP), lambda i: (0, 0)),
                ],
                out_specs=pl.BlockSpec((_MM_BLK, HP), lambda i: (i, 0)),
                out_shape=jax.ShapeDtypeStruct((NBP, HP), F32),
            )(msg, wh_p)
        h = _seg_sum(g, a2b_flat)
        msg = _bond_update(inp, g, h, b2a_p, b2revb_p)

    am = _seg_sum(msg, a2b_flat)

    out1, out2 = pl.pallas_call(
        _readout_body,
        grid=(MOLS // _MB,),
        in_specs=[
            pl.BlockSpec((_AB, AF), lambda i: (i, 0)),
            pl.BlockSpec((_AB, HP), lambda i: (i, 0)),
            pl.BlockSpec((AF, HP), lambda i: (0, 0)),
            pl.BlockSpec((HP, HP), lambda i: (0, 0)),
            pl.BlockSpec((1, HP), lambda i: (0, 0)),
            pl.BlockSpec((_MB, _AB), lambda i: (0, 0)),
        ],
        out_specs=[
            pl.BlockSpec((_MB, A_SIZE, H), lambda i: (i, 0, 0)),
            pl.BlockSpec((_MB, B_SIZE + 1, H), lambda i: (i, 0, 0)),
        ],
        out_shape=[
            jax.ShapeDtypeStruct((MOLS, A_SIZE, H), F32),
            jax.ShapeDtypeStruct((MOLS, B_SIZE + 1, H), F32),
        ],
    )(f_atoms, am, woa_p, wob_p, bo_p, smat)

    return (out1.reshape(NA, H), out2.reshape(NB + MOLS, H))
